# Initial kernel scaffold; baseline (speedup 1.0000x reference)
import jax
import jax.numpy as jnp
from jax.experimental import pallas as pl


def kernel(x):
    raise NotImplementedError("write your pallas kernel here")


# speedup vs baseline: 25.6964x; 25.6964x over previous
import jax
import jax.numpy as jnp
from jax import lax
from jax.experimental import pallas as pl

_ROWS = 8


def _select_body(x_ref, sparse_ref, mask_ref, *, k):
    x = x_ref[...]
    u = lax.bitcast_convert_type(jnp.abs(x), jnp.int32)

    def step(i, p):
        cand = p | (jnp.int32(1) << (30 - i))
        cnt = jnp.sum((u >= cand).astype(jnp.int32), axis=1, keepdims=True)
        return jnp.where(cnt >= k, cand, p)

    p0 = jnp.zeros((x.shape[0], 1), jnp.int32)
    thr = lax.fori_loop(0, 31, step, p0)
    keep = u >= thr
    mask_ref[...] = keep
    sparse_ref[...] = jnp.where(keep, x, 0.0)


def kernel(x):
    flat = x if x.ndim == 2 else x.reshape(x.shape[0], -1)
    B, H = flat.shape
    k = max(1, int(H * 10.0 / 100.0))
    rows = _ROWS if B % _ROWS == 0 else 1
    import functools
    sparse, mask = pl.pallas_call(
        functools.partial(_select_body, k=k),
        grid=(B // rows,),
        in_specs=[pl.BlockSpec((rows, H), lambda i: (i, 0))],
        out_specs=[
            pl.BlockSpec((rows, H), lambda i: (i, 0)),
            pl.BlockSpec((rows, H), lambda i: (i, 0)),
        ],
        out_shape=[
            jax.ShapeDtypeStruct((B, H), jnp.float32),
            jax.ShapeDtypeStruct((B, H), jnp.bool_),
        ],
    )(flat)
    return sparse.reshape(x.shape), mask.reshape(x.shape)
